# Initial kernel scaffold; baseline (speedup 1.0000x reference)
import jax
import jax.numpy as jnp
from jax.experimental import pallas as pl


def kernel(loc, conf, feat0, feat1, feat2, feat3, feat4, feat5):
    raise NotImplementedError("write your pallas kernel here")


# speedup vs baseline: 38.4337x; 38.4337x over previous
import math

import numpy as np
import jax
import jax.numpy as jnp
from jax import lax
from jax.experimental import pallas as pl
from jax.experimental.pallas import tpu as pltpu
from jax.experimental.pallas import tpu_sc as plsc

_IMG = 300
_STEPS = [16, 32, 64, 100, 150, 300]
_MINS = [60, 105, 150, 195, 240, 285]
_MAXS = [105, 150, 195, 240, 285, 330]
_FS = [19, 10, 5, 3, 2, 1]
_ARS = [2, 3]

_B = 8
_P = 3000
_PPAD = 3008
_NCLS = 21
_CAP = 600
_SCAP = 608
_TOPK = 100
_OPAD = 112
_L = 16
_NC, _NS = 2, 16

_NMS_THRESH = 0.45
_SCORE_THRESH = 0.5


def _prior_grid():
    pr = []
    for k, f in enumerate(_FS):
        step = _STEPS[k]
        s = _MINS[k] / _IMG
        sp = math.sqrt(_MINS[k] * _MAXS[k]) / _IMG
        for i in range(f):
            for j in range(f):
                cx = (j + 0.5) * step / _IMG
                cy = (i + 0.5) * step / _IMG
                pr.append([cx, cy, s, s])
                pr.append([cx, cy, sp, sp])
                for ar in _ARS:
                    r = math.sqrt(ar)
                    pr.append([cx, cy, s * r, s / r])
                    pr.append([cx, cy, s / r, s * r])
    a = np.zeros((4, _PPAD), np.float32)
    a[:, :_P] = np.array(pr, np.float32).T
    return a


_PRIORS_NP = _prior_grid()


def _body(conf_hbm, loc_hbm, pri_hbm,
          boxes_out, scores_out, labels_out,
          conf_v, loc_v, pri_v,
          c_score, c_prior, c_label,
          s_score, s_prior, s_label,
          s_x1, s_y1, s_x2, s_y2,
          s_ox1, s_oy1, s_ox2, s_oy2, s_area,
          keep_v,
          o_x1, o_y1, o_x2, o_y2, o_s, o_l):
    wid = lax.axis_index("s") * _NC + lax.axis_index("c")

    @pl.when(wid < _B)
    def _work():
        b = wid
        pltpu.sync_copy(conf_hbm.at[b], conf_v)
        pltpu.sync_copy(loc_hbm.at[b], loc_v)
        pltpu.sync_copy(pri_hbm, pri_v)

        iota = lax.iota(jnp.int32, _L)
        fzero = jnp.zeros((_L,), jnp.float32)
        izero = jnp.zeros((_L,), jnp.int32)

        def z_sorted(c, carry):
            s_score[pl.ds(c * _L, _L)] = fzero
            s_prior[pl.ds(c * _L, _L)] = izero
            s_label[pl.ds(c * _L, _L)] = izero
            return carry

        lax.fori_loop(0, _SCAP // _L, z_sorted, 0)

        def z_out(c, carry):
            o_x1[pl.ds(c * _L, _L)] = fzero
            o_y1[pl.ds(c * _L, _L)] = fzero
            o_x2[pl.ds(c * _L, _L)] = fzero
            o_y2[pl.ds(c * _L, _L)] = fzero
            o_s[pl.ds(c * _L, _L)] = fzero
            o_l[pl.ds(c * _L, _L)] = izero
            return carry

        lax.fori_loop(0, _OPAD // _L, z_out, 0)

        def phase_a(c, cnt):
            base = c * _L
            xs = [conf_v[k, pl.ds(base, _L)] for k in range(_NCLS)]
            m = xs[0]
            for k in range(1, _NCLS):
                m = jnp.maximum(m, xs[k])
            xm1 = xs[1]
            am = jnp.full((_L,), 1, jnp.int32)
            for k in range(2, _NCLS):
                g = xs[k] > xm1
                xm1 = jnp.maximum(xm1, xs[k])
                am = jnp.where(g, k, am)
            ssum = fzero
            for k in range(_NCLS):
                ssum = ssum + jnp.exp(xs[k] - m)
            score = jnp.exp(xm1 - m) / ssum
            valid = score > _SCORE_THRESH
            vi = jnp.where(valid, 1, izero)
            pos = cnt + plsc.cumsum(vi) - vi
            plsc.store_scatter(c_score, [pos], score, mask=valid)
            plsc.store_scatter(c_prior, [pos], base + iota, mask=valid)
            plsc.store_scatter(c_label, [pos], am, mask=valid)
            return cnt + jnp.sum(vi)

        cnt = lax.fori_loop(0, _PPAD // _L, phase_a, jnp.int32(0))

        nc_cand = lax.div(cnt + (_L - 1), jnp.int32(_L))

        def rank_chunk(c, carry):
            base = c * _L
            si = c_score[pl.ds(base, _L)]
            iidx = base + iota

            def cnt_j(j, r):
                sj = plsc.load_gather(c_score, [jnp.full((_L,), j, jnp.int32)])
                hit = (sj > si) | ((sj == si) & (j < iidx))
                return r + jnp.where(hit, 1, izero)

            rank = lax.fori_loop(0, cnt, cnt_j, izero)
            ok = (iidx < cnt) & (rank < _CAP)
            rr = jnp.minimum(rank, _SCAP - 1)
            plsc.store_scatter(s_score, [rr], si, mask=ok)
            plsc.store_scatter(s_prior, [rr], c_prior[pl.ds(base, _L)], mask=ok)
            plsc.store_scatter(s_label, [rr], c_label[pl.ds(base, _L)], mask=ok)
            return carry

        lax.fori_loop(0, nc_cand, rank_chunk, 0)

        v6 = jnp.minimum(cnt, _CAP)
        nch = lax.div(v6 + (_L - 1), jnp.int32(_L))

        def decode(c, carry):
            base = c * _L
            sl = pl.ds(base, _L)
            pidx = s_prior[sl]
            k0 = izero
            k1 = jnp.full((_L,), 1, jnp.int32)
            k2 = jnp.full((_L,), 2, jnp.int32)
            k3 = jnp.full((_L,), 3, jnp.int32)
            pcx = plsc.load_gather(pri_v, [k0, pidx])
            pcy = plsc.load_gather(pri_v, [k1, pidx])
            pw = plsc.load_gather(pri_v, [k2, pidx])
            ph = plsc.load_gather(pri_v, [k3, pidx])
            l0 = plsc.load_gather(loc_v, [k0, pidx])
            l1 = plsc.load_gather(loc_v, [k1, pidx])
            l2 = plsc.load_gather(loc_v, [k2, pidx])
            l3 = plsc.load_gather(loc_v, [k3, pidx])
            cx = pcx + l0 * 0.1 * pw
            cy = pcy + l1 * 0.1 * ph
            w = pw * jnp.exp(l2 * 0.2)
            h = ph * jnp.exp(l3 * 0.2)
            x1 = cx - w * 0.5
            y1 = cy - h * 0.5
            x2 = cx + w * 0.5
            y2 = cy + h * 0.5
            off = s_label[sl].astype(jnp.float32) * 1000.0
            ox1 = x1 + off
            oy1 = y1 + off
            ox2 = x2 + off
            oy2 = y2 + off
            s_x1[sl] = x1
            s_y1[sl] = y1
            s_x2[sl] = x2
            s_y2[sl] = y2
            s_ox1[sl] = ox1
            s_oy1[sl] = oy1
            s_ox2[sl] = ox2
            s_oy2[sl] = oy2
            s_area[sl] = (ox2 - ox1) * (oy2 - oy1)
            keep_v[sl] = jnp.where((base + iota) < v6, 1, izero)
            return carry

        lax.fori_loop(0, nch, decode, 0)

        def nms_i(i, carry):
            ii = jnp.full((_L,), i, jnp.int32)
            ki = plsc.load_gather(keep_v, [ii])
            xi1 = plsc.load_gather(s_ox1, [ii])
            yi1 = plsc.load_gather(s_oy1, [ii])
            xi2 = plsc.load_gather(s_ox2, [ii])
            yi2 = plsc.load_gather(s_oy2, [ii])
            ai = plsc.load_gather(s_area, [ii])
            kflag = ki > 0

            def nms_j(c, c2):
                base = c * _L
                sl = pl.ds(base, _L)
                ltx = jnp.maximum(xi1, s_ox1[sl])
                lty = jnp.maximum(yi1, s_oy1[sl])
                rbx = jnp.minimum(xi2, s_ox2[sl])
                rby = jnp.minimum(yi2, s_oy2[sl])
                ww = jnp.maximum(rbx - ltx, 0.0)
                hh = jnp.maximum(rby - lty, 0.0)
                inter = ww * hh
                iou = inter / (ai + s_area[sl] - inter + 1e-12)
                sup = kflag & (iou > _NMS_THRESH) & ((base + iota) > i)
                keep_v[sl] = jnp.where(sup, 0, keep_v[sl])
                return c2

            lax.fori_loop(lax.div(i, jnp.int32(_L)), nch, nms_j, 0)
            return carry

        lax.fori_loop(0, v6, nms_i, 0)

        def emit(c, bbase):
            base = c * _L
            sl = pl.ds(base, _L)
            kv = keep_v[sl]
            cs = plsc.cumsum(kv)
            pos = bbase + cs - kv
            ok = (kv > 0) & (pos < _TOPK)
            pp = jnp.minimum(pos, _OPAD - 1)
            plsc.store_scatter(o_s, [pp], s_score[sl], mask=ok)
            plsc.store_scatter(o_l, [pp], s_label[sl], mask=ok)
            plsc.store_scatter(o_x1, [pp], jnp.clip(s_x1[sl], 0.0, 1.0), mask=ok)
            plsc.store_scatter(o_y1, [pp], jnp.clip(s_y1[sl], 0.0, 1.0), mask=ok)
            plsc.store_scatter(o_x2, [pp], jnp.clip(s_x2[sl], 0.0, 1.0), mask=ok)
            plsc.store_scatter(o_y2, [pp], jnp.clip(s_y2[sl], 0.0, 1.0), mask=ok)
            return bbase + jnp.sum(kv)

        lax.fori_loop(0, nch, emit, jnp.int32(0))

        pltpu.sync_copy(o_x1, boxes_out.at[b, 0])
        pltpu.sync_copy(o_y1, boxes_out.at[b, 1])
        pltpu.sync_copy(o_x2, boxes_out.at[b, 2])
        pltpu.sync_copy(o_y2, boxes_out.at[b, 3])
        pltpu.sync_copy(o_s, scores_out.at[b])
        pltpu.sync_copy(o_l, labels_out.at[b])


def _build():
    mesh = plsc.VectorSubcoreMesh(core_axis_name="c", subcore_axis_name="s")
    f32, i32 = jnp.float32, jnp.int32
    return pl.kernel(
        _body,
        out_type=(
            jax.ShapeDtypeStruct((_B, 4, _OPAD), f32),
            jax.ShapeDtypeStruct((_B, _OPAD), f32),
            jax.ShapeDtypeStruct((_B, _OPAD), i32),
        ),
        mesh=mesh,
        compiler_params=pltpu.CompilerParams(needs_layout_passes=False),
        scratch_types=[
            pltpu.VMEM((_NCLS, _PPAD), f32),
            pltpu.VMEM((4, _PPAD), f32),
            pltpu.VMEM((4, _PPAD), f32),
            pltpu.VMEM((_PPAD + _L,), f32),
            pltpu.VMEM((_PPAD + _L,), i32),
            pltpu.VMEM((_PPAD + _L,), i32),
            pltpu.VMEM((_SCAP,), f32),
            pltpu.VMEM((_SCAP,), i32),
            pltpu.VMEM((_SCAP,), i32),
            pltpu.VMEM((_SCAP,), f32),
            pltpu.VMEM((_SCAP,), f32),
            pltpu.VMEM((_SCAP,), f32),
            pltpu.VMEM((_SCAP,), f32),
            pltpu.VMEM((_SCAP,), f32),
            pltpu.VMEM((_SCAP,), f32),
            pltpu.VMEM((_SCAP,), f32),
            pltpu.VMEM((_SCAP,), f32),
            pltpu.VMEM((_SCAP,), f32),
            pltpu.VMEM((_SCAP,), i32),
            pltpu.VMEM((_OPAD,), f32),
            pltpu.VMEM((_OPAD,), f32),
            pltpu.VMEM((_OPAD,), f32),
            pltpu.VMEM((_OPAD,), f32),
            pltpu.VMEM((_OPAD,), f32),
            pltpu.VMEM((_OPAD,), i32),
        ],
    )


def kernel(loc, conf, feat0, feat1, feat2, feat3, feat4, feat5):
    conf_t = jnp.pad(jnp.transpose(conf, (0, 2, 1)),
                     ((0, 0), (0, 0), (0, _PPAD - _P)))
    loc_t = jnp.pad(jnp.transpose(loc, (0, 2, 1)),
                    ((0, 0), (0, 0), (0, _PPAD - _P)))
    pri = jnp.asarray(_PRIORS_NP)
    bx, sc, lb = _build()(conf_t, loc_t, pri)
    boxes = jnp.transpose(bx, (0, 2, 1))[:, :_TOPK, :]
    return boxes, sc[:, :_TOPK], lb[:, :_TOPK]


# speedup vs baseline: 46.7144x; 1.2155x over previous
import math

import numpy as np
import jax
import jax.numpy as jnp
from jax import lax
from jax.experimental import pallas as pl
from jax.experimental.pallas import tpu as pltpu
from jax.experimental.pallas import tpu_sc as plsc

_IMG = 300
_STEPS = [16, 32, 64, 100, 150, 300]
_MINS = [60, 105, 150, 195, 240, 285]
_MAXS = [105, 150, 195, 240, 285, 330]
_FS = [19, 10, 5, 3, 2, 1]
_ARS = [2, 3]

_B = 8
_P = 3000
_PPAD = 3008
_Q = 4
_PQ = _PPAD // _Q
_QCAP = _PQ + 16
_NCLS = 21
_CAP = 600
_SCAP = 608
_TOPK = 100
_OPAD = 112
_L = 16
_NC = 2
_BPC = _B // _NC

_NMS_THRESH = 0.45
_SCORE_THRESH = 0.5
_CNTW = 128


def _prior_grid():
    pr = []
    for k, f in enumerate(_FS):
        step = _STEPS[k]
        s = _MINS[k] / _IMG
        sp = math.sqrt(_MINS[k] * _MAXS[k]) / _IMG
        for i in range(f):
            for j in range(f):
                cx = (j + 0.5) * step / _IMG
                cy = (i + 0.5) * step / _IMG
                pr.append([cx, cy, s, s])
                pr.append([cx, cy, sp, sp])
                for ar in _ARS:
                    r = math.sqrt(ar)
                    pr.append([cx, cy, s * r, s / r])
                    pr.append([cx, cy, s / r, s * r])
    a = np.zeros((4, _PPAD), np.float32)
    a[:, :_P] = np.array(pr, np.float32).T
    return a


_PRIORS_NP = _prior_grid()


def _body(conf_hbm, loc_hbm, pri_hbm,
          boxes_out, scores_out, labels_out,
          conf_v, loc_v, pri_v,
          q_score, q_prior, q_label, q_cnt,
          sh_score, sh_prior, sh_label, sh_cnt,
          st_score, st_prior, st_label, st_cnt,
          c_score, c_prior, c_label,
          s_score, s_prior, s_label,
          s_x1, s_y1, s_x2, s_y2,
          s_ox1, s_oy1, s_ox2, s_oy2, s_area,
          keep_v,
          o_x1, o_y1, o_x2, o_y2, o_s, o_l):
    core = lax.axis_index("c")
    sub = lax.axis_index("s")
    lb = lax.div(sub, jnp.int32(_Q))
    q = lax.rem(sub, jnp.int32(_Q))
    b = core * _BPC + lb

    iota = lax.iota(jnp.int32, _L)
    fzero = jnp.zeros((_L,), jnp.float32)
    izero = jnp.zeros((_L,), jnp.int32)

    pltpu.sync_copy(conf_hbm.at[b * _Q + q], conf_v)

    @pl.when(q == 0)
    def _leader_loads():
        pltpu.sync_copy(loc_hbm.at[b], loc_v)
        pltpu.sync_copy(pri_hbm, pri_v)

    pbase = q * _PQ

    def phase_a(c, cnt):
        base = c * _L
        xs = [conf_v[k, pl.ds(base, _L)] for k in range(_NCLS)]
        m = xs[0]
        for k in range(1, _NCLS):
            m = jnp.maximum(m, xs[k])
        xm1 = xs[1]
        am = jnp.full((_L,), 1, jnp.int32)
        for k in range(2, _NCLS):
            g = xs[k] > xm1
            xm1 = jnp.maximum(xm1, xs[k])
            am = jnp.where(g, k, am)
        ssum = fzero
        for k in range(_NCLS):
            ssum = ssum + jnp.exp(xs[k] - m)
        score = jnp.exp(xm1 - m) / ssum
        valid = score > _SCORE_THRESH
        vi = jnp.where(valid, 1, izero)
        pos = cnt + plsc.cumsum(vi) - vi
        plsc.store_scatter(q_score, [pos], score, mask=valid)
        plsc.store_scatter(q_prior, [pos], pbase + base + iota, mask=valid)
        plsc.store_scatter(q_label, [pos], am, mask=valid)
        return cnt + jnp.sum(vi)

    cntq = lax.fori_loop(0, _PQ // _L, phase_a, jnp.int32(0))
    def z_cnt(c, carry):
        q_cnt[pl.ds(c * _L, _L)] = jnp.full((_L,), cntq, jnp.int32)
        return carry

    lax.fori_loop(0, _CNTW // _L, z_cnt, 0)

    pltpu.sync_copy(q_score, sh_score.at[sub])
    pltpu.sync_copy(q_prior, sh_prior.at[sub])
    pltpu.sync_copy(q_label, sh_label.at[sub])
    pltpu.sync_copy(q_cnt, sh_cnt.at[sub])

    plsc.subcore_barrier()

    @pl.when(q == 0)
    def _leader():
        for qq in range(_Q):
            pltpu.sync_copy(sh_score.at[sub + qq], st_score.at[qq])
            pltpu.sync_copy(sh_prior.at[sub + qq], st_prior.at[qq])
            pltpu.sync_copy(sh_label.at[sub + qq], st_label.at[qq])
            pltpu.sync_copy(sh_cnt.at[sub + qq], st_cnt.at[qq])

        def z_sorted(c, carry):
            s_score[pl.ds(c * _L, _L)] = fzero
            s_prior[pl.ds(c * _L, _L)] = izero
            s_label[pl.ds(c * _L, _L)] = izero
            return carry

        lax.fori_loop(0, _SCAP // _L, z_sorted, 0)

        def z_out(c, carry):
            o_x1[pl.ds(c * _L, _L)] = fzero
            o_y1[pl.ds(c * _L, _L)] = fzero
            o_x2[pl.ds(c * _L, _L)] = fzero
            o_y2[pl.ds(c * _L, _L)] = fzero
            o_s[pl.ds(c * _L, _L)] = fzero
            o_l[pl.ds(c * _L, _L)] = izero
            return carry

        lax.fori_loop(0, _OPAD // _L, z_out, 0)

        cq = [jnp.clip(st_cnt[qq, pl.ds(0, _L)][0], 0, _PQ) for qq in range(_Q)]

        def merge_one(qq, base_off):
            nqc = lax.div(cq[qq] + (_L - 1), jnp.int32(_L))

            def mv(ch, carry):
                loff = ch * _L
                lidx = loff + iota
                ok = lidx < cq[qq]
                pos = jnp.minimum(base_off + lidx, _PPAD + _L - 1)
                plsc.store_scatter(c_score, [pos], st_score[qq, pl.ds(loff, _L)], mask=ok)
                plsc.store_scatter(c_prior, [pos], st_prior[qq, pl.ds(loff, _L)], mask=ok)
                plsc.store_scatter(c_label, [pos], st_label[qq, pl.ds(loff, _L)], mask=ok)
                return carry

            lax.fori_loop(0, nqc, mv, 0)
            return base_off + cq[qq]

        cnt = jnp.int32(0)
        for qq in range(_Q):
            cnt = merge_one(qq, cnt)

        nc_cand = lax.div(cnt + (_L - 1), jnp.int32(_L))

        def rank_chunk(c, carry):
            base = c * _L
            si = c_score[pl.ds(base, _L)]
            iidx = base + iota

            def cnt_j(j, r):
                sj = plsc.load_gather(c_score, [jnp.full((_L,), j, jnp.int32)])
                hit = (sj > si) | ((sj == si) & (j < iidx))
                return r + jnp.where(hit, 1, izero)

            rank = lax.fori_loop(0, cnt, cnt_j, izero)
            ok = (iidx < cnt) & (rank < _CAP)
            rr = jnp.minimum(rank, _SCAP - 1)
            plsc.store_scatter(s_score, [rr], si, mask=ok)
            plsc.store_scatter(s_prior, [rr], c_prior[pl.ds(base, _L)], mask=ok)
            plsc.store_scatter(s_label, [rr], c_label[pl.ds(base, _L)], mask=ok)
            return carry

        lax.fori_loop(0, nc_cand, rank_chunk, 0)

        v6 = jnp.minimum(cnt, _CAP)
        nch = lax.div(v6 + (_L - 1), jnp.int32(_L))

        def decode(c, carry):
            base = c * _L
            sl = pl.ds(base, _L)
            pidx = s_prior[sl]
            k0 = izero
            k1 = jnp.full((_L,), 1, jnp.int32)
            k2 = jnp.full((_L,), 2, jnp.int32)
            k3 = jnp.full((_L,), 3, jnp.int32)
            pcx = plsc.load_gather(pri_v, [k0, pidx])
            pcy = plsc.load_gather(pri_v, [k1, pidx])
            pw = plsc.load_gather(pri_v, [k2, pidx])
            ph = plsc.load_gather(pri_v, [k3, pidx])
            l0 = plsc.load_gather(loc_v, [k0, pidx])
            l1 = plsc.load_gather(loc_v, [k1, pidx])
            l2 = plsc.load_gather(loc_v, [k2, pidx])
            l3 = plsc.load_gather(loc_v, [k3, pidx])
            cx = pcx + l0 * 0.1 * pw
            cy = pcy + l1 * 0.1 * ph
            w = pw * jnp.exp(l2 * 0.2)
            h = ph * jnp.exp(l3 * 0.2)
            x1 = cx - w * 0.5
            y1 = cy - h * 0.5
            x2 = cx + w * 0.5
            y2 = cy + h * 0.5
            off = s_label[sl].astype(jnp.float32) * 1000.0
            ox1 = x1 + off
            oy1 = y1 + off
            ox2 = x2 + off
            oy2 = y2 + off
            s_x1[sl] = x1
            s_y1[sl] = y1
            s_x2[sl] = x2
            s_y2[sl] = y2
            s_ox1[sl] = ox1
            s_oy1[sl] = oy1
            s_ox2[sl] = ox2
            s_oy2[sl] = oy2
            s_area[sl] = (ox2 - ox1) * (oy2 - oy1)
            keep_v[sl] = jnp.where((base + iota) < v6, 1, izero)
            return carry

        lax.fori_loop(0, nch, decode, 0)

        def nms_i(i, carry):
            ii = jnp.full((_L,), i, jnp.int32)
            ki = plsc.load_gather(keep_v, [ii])
            xi1 = plsc.load_gather(s_ox1, [ii])
            yi1 = plsc.load_gather(s_oy1, [ii])
            xi2 = plsc.load_gather(s_ox2, [ii])
            yi2 = plsc.load_gather(s_oy2, [ii])
            ai = plsc.load_gather(s_area, [ii])
            kflag = ki > 0

            def nms_j(c, c2):
                base = c * _L
                sl = pl.ds(base, _L)
                ltx = jnp.maximum(xi1, s_ox1[sl])
                lty = jnp.maximum(yi1, s_oy1[sl])
                rbx = jnp.minimum(xi2, s_ox2[sl])
                rby = jnp.minimum(yi2, s_oy2[sl])
                ww = jnp.maximum(rbx - ltx, 0.0)
                hh = jnp.maximum(rby - lty, 0.0)
                inter = ww * hh
                iou = inter / (ai + s_area[sl] - inter + 1e-12)
                sup = kflag & (iou > _NMS_THRESH) & ((base + iota) > i)
                keep_v[sl] = jnp.where(sup, 0, keep_v[sl])
                return c2

            lax.fori_loop(lax.div(i, jnp.int32(_L)), nch, nms_j, 0)
            return carry

        lax.fori_loop(0, v6, nms_i, 0)

        def emit(c, bbase):
            base = c * _L
            sl = pl.ds(base, _L)
            kv = keep_v[sl]
            cs = plsc.cumsum(kv)
            pos = bbase + cs - kv
            ok = (kv > 0) & (pos < _TOPK)
            pp = jnp.minimum(pos, _OPAD - 1)
            plsc.store_scatter(o_s, [pp], s_score[sl], mask=ok)
            plsc.store_scatter(o_l, [pp], s_label[sl], mask=ok)
            plsc.store_scatter(o_x1, [pp], jnp.clip(s_x1[sl], 0.0, 1.0), mask=ok)
            plsc.store_scatter(o_y1, [pp], jnp.clip(s_y1[sl], 0.0, 1.0), mask=ok)
            plsc.store_scatter(o_x2, [pp], jnp.clip(s_x2[sl], 0.0, 1.0), mask=ok)
            plsc.store_scatter(o_y2, [pp], jnp.clip(s_y2[sl], 0.0, 1.0), mask=ok)
            return bbase + jnp.sum(kv)

        lax.fori_loop(0, nch, emit, jnp.int32(0))

        pltpu.sync_copy(o_x1, boxes_out.at[b, 0])
        pltpu.sync_copy(o_y1, boxes_out.at[b, 1])
        pltpu.sync_copy(o_x2, boxes_out.at[b, 2])
        pltpu.sync_copy(o_y2, boxes_out.at[b, 3])
        pltpu.sync_copy(o_s, scores_out.at[b])
        pltpu.sync_copy(o_l, labels_out.at[b])


def _build():
    mesh = plsc.VectorSubcoreMesh(core_axis_name="c", subcore_axis_name="s")
    f32, i32 = jnp.float32, jnp.int32
    return pl.kernel(
        _body,
        out_type=(
            jax.ShapeDtypeStruct((_B, 4, _OPAD), f32),
            jax.ShapeDtypeStruct((_B, _OPAD), f32),
            jax.ShapeDtypeStruct((_B, _OPAD), i32),
        ),
        mesh=mesh,
        compiler_params=pltpu.CompilerParams(needs_layout_passes=False),
        scratch_types=[
            pltpu.VMEM((_NCLS, _PQ), f32),
            pltpu.VMEM((4, _PPAD), f32),
            pltpu.VMEM((4, _PPAD), f32),
            pltpu.VMEM((_QCAP,), f32),
            pltpu.VMEM((_QCAP,), i32),
            pltpu.VMEM((_QCAP,), i32),
            pltpu.VMEM((_CNTW,), i32),
            pltpu.VMEM_SHARED((_BPC * _Q, _QCAP), f32),
            pltpu.VMEM_SHARED((_BPC * _Q, _QCAP), i32),
            pltpu.VMEM_SHARED((_BPC * _Q, _QCAP), i32),
            pltpu.VMEM_SHARED((_BPC * _Q, _CNTW), i32),
            pltpu.VMEM((_Q, _QCAP), f32),
            pltpu.VMEM((_Q, _QCAP), i32),
            pltpu.VMEM((_Q, _QCAP), i32),
            pltpu.VMEM((_Q, _CNTW), i32),
            pltpu.VMEM((_PPAD + _L,), f32),
            pltpu.VMEM((_PPAD + _L,), i32),
            pltpu.VMEM((_PPAD + _L,), i32),
            pltpu.VMEM((_SCAP,), f32),
            pltpu.VMEM((_SCAP,), i32),
            pltpu.VMEM((_SCAP,), i32),
            pltpu.VMEM((_SCAP,), f32),
            pltpu.VMEM((_SCAP,), f32),
            pltpu.VMEM((_SCAP,), f32),
            pltpu.VMEM((_SCAP,), f32),
            pltpu.VMEM((_SCAP,), f32),
            pltpu.VMEM((_SCAP,), f32),
            pltpu.VMEM((_SCAP,), f32),
            pltpu.VMEM((_SCAP,), f32),
            pltpu.VMEM((_SCAP,), f32),
            pltpu.VMEM((_SCAP,), i32),
            pltpu.VMEM((_OPAD,), f32),
            pltpu.VMEM((_OPAD,), f32),
            pltpu.VMEM((_OPAD,), f32),
            pltpu.VMEM((_OPAD,), f32),
            pltpu.VMEM((_OPAD,), f32),
            pltpu.VMEM((_OPAD,), i32),
        ],
    )


def kernel(loc, conf, feat0, feat1, feat2, feat3, feat4, feat5):
    conf_t = jnp.pad(jnp.transpose(conf, (0, 2, 1)),
                     ((0, 0), (0, 0), (0, _PPAD - _P)))
    conf_q = jnp.transpose(conf_t.reshape(_B, _NCLS, _Q, _PQ),
                           (0, 2, 1, 3)).reshape(_B * _Q, _NCLS, _PQ)
    loc_t = jnp.pad(jnp.transpose(loc, (0, 2, 1)),
                    ((0, 0), (0, 0), (0, _PPAD - _P)))
    pri = jnp.asarray(_PRIORS_NP)
    bx, sc, lb = _build()(conf_q, loc_t, pri)
    boxes = jnp.transpose(bx, (0, 2, 1))[:, :_TOPK, :]
    return boxes, sc[:, :_TOPK], lb[:, :_TOPK]
